# Initial kernel scaffold; baseline (speedup 1.0000x reference)
import jax
import jax.numpy as jnp
from jax.experimental import pallas as pl


def kernel(x, cat_features, edge_index, batch, W1, a_src1, a_dst1, b1, W_rel, b_rel, W_root, Wl3, bl3, Wr3, br3, att3, b3, Wl4, bl4, Wr4, br4, att4, b4, Wfc1, bfc1, Wfc2, bfc2):
    raise NotImplementedError("write your pallas kernel here")


# speedup vs baseline: 3.1374x; 3.1374x over previous
import functools

import jax
import jax.numpy as jnp
from jax import lax
from jax.experimental import pallas as pl
from jax.experimental.pallas import tpu as pltpu
from jax.experimental.pallas import tpu_sc as plsc

N = 10000
E = 320000
H = 128
NCAT = 16
NG = 64
FCH = 600

NCORES = 2
NSUB = 16
NW = NCORES * NSUB
EPT = E // NW
K = 80
NCHUNK = EPT // K
NWRITE = 10
RW = N // NWRITE
ZR = 40
F32 = jnp.float32
I32 = jnp.int32


def _leaky(t):
    return jnp.where(t > 0, t, 0.2 * t)


def _tc1_body(x_ref, w1_ref, asrc_ref, adst_ref, xl_ref, s_ref, d_ref):
    xl = jnp.dot(x_ref[...], w1_ref[...], preferred_element_type=F32)
    xl_ref[...] = xl
    s_ref[...] = jnp.sum(xl * asrc_ref[...][None, :], axis=1, keepdims=True)
    d_ref[...] = jnp.sum(xl * adst_ref[...][None, :], axis=1, keepdims=True)


def _tc2_body(u_ref, sa_ref, xl_ref, s_ref, d_ref, b1_ref, h_ref):
    w = jnp.exp(_leaky(s_ref[...] + d_ref[...]))
    xl = xl_ref[...]
    U = u_ref[0] + u_ref[1] + w * xl
    S = sa_ref[0] + sa_ref[1] + w
    h_ref[...] = jnp.maximum(U / jnp.maximum(S, 1e-16) + b1_ref[...][None, :], 0.0)


def _tc3_body(agg_ref, h1_ref, wrel_ref, brel_ref, wroot_ref,
              wl_ref, bl_ref, wr_ref, br_ref, att_ref,
              h2_ref, zl_ref, zr_ref, w3_ref):
    agg = agg_ref[0] + agg_ref[1]
    h2 = jnp.maximum(
        jnp.dot(agg, wrel_ref[...], preferred_element_type=F32)
        + brel_ref[...][None, :]
        + jnp.dot(h1_ref[...], wroot_ref[...], preferred_element_type=F32), 0.0)
    zl = jnp.dot(h2, wl_ref[...], preferred_element_type=F32) + bl_ref[...][None, :]
    zr = jnp.dot(h2, wr_ref[...], preferred_element_type=F32) + br_ref[...][None, :]
    h2_ref[...] = h2
    zl_ref[...] = zl
    zr_ref[...] = zr
    w3_ref[...] = jnp.exp(jnp.dot(_leaky(zl + zr), att_ref[...],
                                  preferred_element_type=F32))


def _tc4_body(u_ref, sa_ref, zl3_ref, w3_ref, b3_ref, h2_ref,
              wl_ref, bl_ref, wr_ref, br_ref, att_ref,
              zl_ref, zr_ref, w4_ref):
    w3 = w3_ref[...]
    U = u_ref[0] + u_ref[1] + w3 * zl3_ref[...]
    S = sa_ref[0] + sa_ref[1] + w3
    h3 = jnp.maximum(U / jnp.maximum(S, 1e-16) + b3_ref[...][None, :], 0.0)
    h = h3 + h2_ref[...]
    zl = jnp.dot(h, wl_ref[...], preferred_element_type=F32) + bl_ref[...][None, :]
    zr = jnp.dot(h, wr_ref[...], preferred_element_type=F32) + br_ref[...][None, :]
    zl_ref[...] = zl
    zr_ref[...] = zr
    w4_ref[...] = jnp.exp(jnp.dot(_leaky(zl + zr), att_ref[...],
                                  preferred_element_type=F32))


def _tc5_body(u_ref, sa_ref, zl4_ref, w4_ref, b4_ref, batch_ref, cat_ref,
              wfc1_ref, bfc1_ref, wfc2_ref, bfc2_ref, y_ref):
    w4 = w4_ref[...]
    U = u_ref[0] + u_ref[1] + w4 * zl4_ref[...]
    S = sa_ref[0] + sa_ref[1] + w4
    h = jnp.maximum(U / jnp.maximum(S, 1e-16) + b4_ref[...][None, :], 0.0)
    onehot = (batch_ref[...] == lax.broadcasted_iota(I32, (N, NG), 1)).astype(F32)
    pooled = lax.dot_general(onehot, h, (((0,), (0,)), ((), ())),
                             preferred_element_type=F32)
    z = jnp.concatenate([pooled, cat_ref[...]], axis=1)
    z = jnp.maximum(jnp.dot(z, wfc1_ref[...], preferred_element_type=F32)
                    + bfc1_ref[...][None, :], 0.0)
    y_ref[...] = jnp.dot(z, wfc2_ref[...], preferred_element_type=F32) \
        + bfc2_ref[...][None, :]


def _tc(body, out_shapes, *args):
    return pl.pallas_call(body, out_shape=out_shapes)(*args)


def _make_edge_pass(mode):
    gat1 = mode == "gat1"
    gatv2 = mode == "gatv2"
    conv = mode == "conv"

    mesh = plsc.VectorSubcoreMesh(core_axis_name="c", subcore_axis_name="s")

    outs = [jax.ShapeDtypeStruct((NCORES, N, H), F32)]
    if not conv:
        outs.append(jax.ShapeDtypeStruct((NCORES, N), F32))

    scratch = [
        pltpu.VMEM((K,), I32),
        pltpu.VMEM((K,), I32),
        pltpu.VMEM((K, H), F32),
    ]
    if gatv2:
        scratch.append(pltpu.VMEM((K, H), F32))
    scratch += [
        pltpu.VMEM((ZR, H), F32),
        pltpu.VMEM_SHARED((N, H), F32),
    ]
    if not conv:
        scratch += [
            pltpu.VMEM((K,), F32),
            pltpu.VMEM((2000,), F32),
            pltpu.VMEM_SHARED((N,), F32),
        ]
    if gat1:
        scratch += [pltpu.VMEM((N,), F32), pltpu.VMEM((N,), F32)]
    if gatv2:
        scratch.append(pltpu.VMEM((H,), F32))
    scratch.append(pltpu.SemaphoreType.DMA)

    def body(*refs):
        it = iter(refs)
        src_hbm = next(it)
        dst_hbm = next(it)
        feat_hbm = next(it)
        featR_hbm = next(it) if gatv2 else None
        s_hbm = next(it) if gat1 else None
        d_hbm = next(it) if gat1 else None
        att_hbm = next(it) if gatv2 else None
        outU = next(it)
        outS = None if conv else next(it)
        idx_s = next(it)
        idx_d = next(it)
        rowsL = next(it)
        rowsR = next(it) if gatv2 else None
        zbuf = next(it)
        sharedU = next(it)
        if not conv:
            wbuf = next(it)
            zs = next(it)
            sharedS = next(it)
        if gat1:
            sv = next(it)
            dv = next(it)
        if gatv2:
            attv = next(it)
        sem = next(it)

        cid = lax.axis_index("c")
        sid = lax.axis_index("s")
        wid = sid * NCORES + cid
        ebase = wid * EPT

        zero16 = jnp.zeros((16,), F32)
        for r in range(ZR):
            for j in range(H // 16):
                zbuf[r, pl.ds(j * 16, 16)] = zero16
        r0 = pl.multiple_of(sid * RW, 8)
        @pl.when(sid < NWRITE)
        def _zero_u():
            def zcp(q, _):
                pltpu.sync_copy(zbuf, sharedU.at[pl.ds(r0 + q * ZR, ZR), :])
                return 0
            lax.fori_loop(0, RW // ZR, zcp, 0)
        if not conv:
            def zs_(i, _):
                zs[pl.ds(i * 16, 16)] = zero16
                return 0
            lax.fori_loop(0, 125, zs_, 0)
            @pl.when(sid == 0)
            def _():
                def scp(q, _):
                    pltpu.sync_copy(zs, sharedS.at[pl.ds(q * 2000, 2000)])
                    return 0
                lax.fori_loop(0, N // 2000, scp, 0)

        if gat1:
            pltpu.sync_copy(s_hbm, sv)
            pltpu.sync_copy(d_hbm, dv)
        if gatv2:
            pltpu.sync_copy(att_hbm, attv)

        plsc.subcore_barrier()

        def chunk(ci, _):
            eb = ebase + ci * K
            pltpu.sync_copy(src_hbm.at[pl.ds(eb, K)], idx_s)
            pltpu.sync_copy(dst_hbm.at[pl.ds(eb, K)], idx_d)
            pltpu.async_copy(feat_hbm.at[idx_s], rowsL, sem).wait()
            if gatv2:
                pltpu.async_copy(featR_hbm.at[idx_d], rowsR, sem).wait()

            if gat1:
                def wg(g, _):
                    sidx = idx_s[pl.ds(g * 16, 16)]
                    didx = idx_d[pl.ds(g * 16, 16)]
                    t = plsc.load_gather(sv, [sidx]) + plsc.load_gather(dv, [didx])
                    wbuf[pl.ds(g * 16, 16)] = jnp.exp(_leaky(t))
                    return 0
                lax.fori_loop(0, K // 16, wg, 0)
            if gatv2:
                def wg(g, _):
                    e16 = lax.iota(I32, 16) + g * 16
                    def lf(fb, acc):
                        av = attv[pl.ds(fb * 16, 16)]
                        for j in range(16):
                            f16 = jnp.full((16,), fb * 16 + j, I32)
                            t = plsc.load_gather(rowsL, [e16, f16]) \
                                + plsc.load_gather(rowsR, [e16, f16])
                            acc = acc + av[j] * _leaky(t)
                        return acc
                    acc = lax.fori_loop(0, H // 16, lf, jnp.zeros((16,), F32))
                    wbuf[pl.ds(g * 16, 16)] = jnp.exp(acc)
                    return 0
                lax.fori_loop(0, K // 16, wg, 0)

            if not conv:
                def sc_(g, _):
                    e16 = lax.iota(I32, 16) + g * 16
                    w16 = wbuf[pl.ds(g * 16, 16)]
                    def sf(fb, _):
                        for j in range(4):
                            f16 = jnp.full((16,), fb * 4 + j, I32)
                            v = plsc.load_gather(rowsL, [e16, f16])
                            plsc.store_scatter(rowsL, [e16, f16], v * w16)
                        return 0
                    lax.fori_loop(0, H // 4, sf, 0)
                    return 0
                lax.fori_loop(0, K // 16, sc_, 0)
                pltpu.sync_copy(wbuf, sharedS.at[idx_d], add=True)

            pltpu.sync_copy(rowsL, sharedU.at[idx_d], add=True)
            return 0
        lax.fori_loop(0, NCHUNK, chunk, 0)

        plsc.subcore_barrier()
        @pl.when(sid < NWRITE)
        def _write_u():
            pltpu.sync_copy(sharedU.at[pl.ds(r0, RW), :],
                            outU.at[cid, pl.ds(r0, RW), :])
        if not conv:
            @pl.when(sid == 0)
            def _write_s():
                pltpu.sync_copy(sharedS, outS.at[cid])

    return pl.kernel(body, out_type=tuple(outs) if len(outs) > 1 else outs[0],
                     mesh=mesh, scratch_types=scratch,
                     compiler_params=pltpu.CompilerParams(
                         needs_layout_passes=False))


_gat1_pass = None
_conv_pass = None
_gatv2_pass = None


def _edge_passes():
    global _gat1_pass, _conv_pass, _gatv2_pass
    if _gat1_pass is None:
        _gat1_pass = _make_edge_pass("gat1")
        _conv_pass = _make_edge_pass("conv")
        _gatv2_pass = _make_edge_pass("gatv2")
    return _gat1_pass, _conv_pass, _gatv2_pass


def kernel(x, cat_features, edge_index, batch, W1, a_src1, a_dst1, b1,
           W_rel, b_rel, W_root, Wl3, bl3, Wr3, br3, att3, b3,
           Wl4, bl4, Wr4, br4, att4, b4, Wfc1, bfc1, Wfc2, bfc2):
    gat1_pass, conv_pass, gatv2_pass = _edge_passes()
    src = edge_index[0]
    dst = edge_index[1]

    sN = jax.ShapeDtypeStruct((N, H), F32)
    s1 = jax.ShapeDtypeStruct((N, 1), F32)

    xl, s, d = _tc(_tc1_body, [sN, s1, s1], x, W1, a_src1, a_dst1)
    U1, S1 = gat1_pass(src, dst, xl, s.reshape(N), d.reshape(N))
    h1 = _tc(_tc2_body, sN, U1, S1.reshape(NCORES, N, 1), xl, s, d, b1)
    A2 = conv_pass(src, dst, h1)
    h2, zl3, zr3, w3 = _tc(
        _tc3_body, [sN, sN, sN, s1],
        A2, h1, W_rel, b_rel, W_root, Wl3, bl3, Wr3, br3, att3.reshape(H, 1))
    U3, S3 = gatv2_pass(src, dst, zl3, zr3, att3)
    zl4, zr4, w4 = _tc(
        _tc4_body, [sN, sN, s1],
        U3, S3.reshape(NCORES, N, 1), zl3, w3, b3, h2,
        Wl4, bl4, Wr4, br4, att4.reshape(H, 1))
    U4, S4 = gatv2_pass(src, dst, zl4, zr4, att4)
    y = _tc(
        _tc5_body, jax.ShapeDtypeStruct((NG, 1), F32),
        U4, S4.reshape(NCORES, N, 1), zl4, w4, b4, batch.reshape(N, 1),
        cat_features, Wfc1, bfc1, Wfc2, bfc2)
    return y


# speedup vs baseline: 7.8267x; 2.4946x over previous
import jax
import jax.numpy as jnp
from jax import lax
from jax.experimental import pallas as pl
from jax.experimental.pallas import tpu as pltpu
from jax.experimental.pallas import tpu_sc as plsc

N = 10000
E = 320000
H = 128
NCAT = 16
NG = 64
FCH = 600

NCORES = 2
NSUB = 16
NW = NCORES * NSUB
EPT = E // NW
K = 80
NCHUNK = EPT // K
NWRITE = 10
RW = N // NWRITE
ZR = 40
BE = 8000
F32 = jnp.float32
I32 = jnp.int32


def _leaky(t):
    return jnp.where(t > 0, t, 0.2 * t)


def _tc1_body(x_ref, w1_ref, asrc_ref, adst_ref, xl_ref, s_ref, d_ref):
    xl = jnp.dot(x_ref[...], w1_ref[...], preferred_element_type=F32)
    xl_ref[...] = xl
    s_ref[...] = jnp.sum(xl * asrc_ref[...][None, :], axis=1, keepdims=True)
    d_ref[...] = jnp.sum(xl * adst_ref[...][None, :], axis=1, keepdims=True)


def _tc2_body(u_ref, sa_ref, xl_ref, s_ref, d_ref, b1_ref, h_ref):
    w = jnp.exp(_leaky(s_ref[...] + d_ref[...]))
    xl = xl_ref[...]
    U = u_ref[0] + u_ref[1] + w * xl
    S = sa_ref[0] + sa_ref[1] + w
    h_ref[...] = jnp.maximum(U / jnp.maximum(S, 1e-16) + b1_ref[...][None, :], 0.0)


def _tc3_body(agg_ref, h1_ref, wrel_ref, brel_ref, wroot_ref,
              wl_ref, bl_ref, wr_ref, br_ref, att_ref,
              h2_ref, zl_ref, zr_ref, w3_ref):
    agg = agg_ref[0] + agg_ref[1]
    h2 = jnp.maximum(
        jnp.dot(agg, wrel_ref[...], preferred_element_type=F32)
        + brel_ref[...][None, :]
        + jnp.dot(h1_ref[...], wroot_ref[...], preferred_element_type=F32), 0.0)
    zl = jnp.dot(h2, wl_ref[...], preferred_element_type=F32) + bl_ref[...][None, :]
    zr = jnp.dot(h2, wr_ref[...], preferred_element_type=F32) + br_ref[...][None, :]
    h2_ref[...] = h2
    zl_ref[...] = zl
    zr_ref[...] = zr
    w3_ref[...] = jnp.exp(jnp.sum(_leaky(zl + zr) * att_ref[...],
                                  axis=1, keepdims=True))


def _tc4_body(u_ref, sa_ref, zl3_ref, w3_ref, b3_ref, h2_ref,
              wl_ref, bl_ref, wr_ref, br_ref, att_ref,
              zl_ref, zr_ref, w4_ref):
    w3 = w3_ref[...]
    U = u_ref[0] + u_ref[1] + w3 * zl3_ref[...]
    S = sa_ref[0] + sa_ref[1] + w3
    h3 = jnp.maximum(U / jnp.maximum(S, 1e-16) + b3_ref[...][None, :], 0.0)
    h = h3 + h2_ref[...]
    zl = jnp.dot(h, wl_ref[...], preferred_element_type=F32) + bl_ref[...][None, :]
    zr = jnp.dot(h, wr_ref[...], preferred_element_type=F32) + br_ref[...][None, :]
    zl_ref[...] = zl
    zr_ref[...] = zr
    w4_ref[...] = jnp.exp(jnp.sum(_leaky(zl + zr) * att_ref[...],
                                  axis=1, keepdims=True))


def _tc5_body(u_ref, sa_ref, zl4_ref, w4_ref, b4_ref, batch_ref, cat_ref,
              wfc1_ref, bfc1_ref, wfc2_ref, bfc2_ref, y_ref):
    w4 = w4_ref[...]
    U = u_ref[0] + u_ref[1] + w4 * zl4_ref[...]
    S = sa_ref[0] + sa_ref[1] + w4
    h = jnp.maximum(U / jnp.maximum(S, 1e-16) + b4_ref[...][None, :], 0.0)
    onehot = (batch_ref[...] == lax.broadcasted_iota(I32, (N, NG), 1)).astype(F32)
    pooled = lax.dot_general(onehot, h, (((0,), (0,)), ((), ())),
                             preferred_element_type=F32)
    z = jnp.concatenate([pooled, cat_ref[...]], axis=1)
    z = jnp.maximum(jnp.dot(z, wfc1_ref[...], preferred_element_type=F32)
                    + bfc1_ref[...][None, :], 0.0)
    y_ref[...] = jnp.dot(z, wfc2_ref[...], preferred_element_type=F32) \
        + bfc2_ref[...][None, :]


def _tc(body, out_shapes, *args):
    return pl.pallas_call(body, out_shape=out_shapes)(*args)


def _tcs_gatv2_body(l_ref, r_ref, att_ref, wl_ref, w_ref):
    L = l_ref[...]
    w = jnp.exp(jnp.sum(_leaky(L + r_ref[...]) * att_ref[...],
                        axis=1, keepdims=True))
    w_ref[...] = w
    wl_ref[...] = w * L


def _tcs_gat1_body(l_ref, w_ref, wl_ref):
    wl_ref[...] = w_ref[...] * l_ref[...]


def _tc_scale_gatv2(Lst, Rst, att):
    blk = lambda i: (i, 0)
    return pl.pallas_call(
        _tcs_gatv2_body,
        grid=(E // BE,),
        in_specs=[pl.BlockSpec((BE, H), blk), pl.BlockSpec((BE, H), blk),
                  pl.BlockSpec((1, H), lambda i: (0, 0))],
        out_specs=[pl.BlockSpec((BE, H), blk), pl.BlockSpec((BE, 1), blk)],
        out_shape=[jax.ShapeDtypeStruct((E, H), F32),
                   jax.ShapeDtypeStruct((E, 1), F32)],
    )(Lst, Rst, att)


def _tc_scale_gat1(Lst, wst):
    blk = lambda i: (i, 0)
    return pl.pallas_call(
        _tcs_gat1_body,
        grid=(E // BE,),
        in_specs=[pl.BlockSpec((BE, H), blk), pl.BlockSpec((BE, 1), blk)],
        out_specs=pl.BlockSpec((BE, H), blk),
        out_shape=jax.ShapeDtypeStruct((E, H), F32),
    )(Lst, wst)


_MESH = plsc.VectorSubcoreMesh(core_axis_name="c", subcore_axis_name="s")
_SC_PARAMS = pltpu.CompilerParams(needs_layout_passes=False)


def _wid_base():
    cid = lax.axis_index("c")
    sid = lax.axis_index("s")
    return cid, sid, (sid * NCORES + cid) * EPT


def _gather2_body(src_hbm, dst_hbm, fl_hbm, fr_hbm, lst_hbm, rst_hbm,
                  idx_s, idx_d, rowsL, rowsR, sem):
    _, _, ebase = _wid_base()

    def chunk(ci, _):
        eb = ebase + ci * K
        pltpu.sync_copy(src_hbm.at[pl.ds(eb, K)], idx_s)
        pltpu.sync_copy(dst_hbm.at[pl.ds(eb, K)], idx_d)
        cl = pltpu.async_copy(fl_hbm.at[idx_s], rowsL, sem)
        cl.wait()
        cr = pltpu.async_copy(fr_hbm.at[idx_d], rowsR, sem)
        cr.wait()
        pltpu.sync_copy(rowsL, lst_hbm.at[pl.ds(eb, K), :])
        pltpu.sync_copy(rowsR, rst_hbm.at[pl.ds(eb, K), :])
        return 0
    lax.fori_loop(0, NCHUNK, chunk, 0)


def _gather2_pass(src, dst, featL, featR):
    return pl.kernel(
        _gather2_body,
        out_type=(jax.ShapeDtypeStruct((E, H), F32),
                  jax.ShapeDtypeStruct((E, H), F32)),
        mesh=_MESH,
        scratch_types=[pltpu.VMEM((K,), I32), pltpu.VMEM((K,), I32),
                       pltpu.VMEM((K, H), F32), pltpu.VMEM((K, H), F32),
                       pltpu.SemaphoreType.DMA],
        compiler_params=_SC_PARAMS,
    )(src, dst, featL, featR)


def _gat1_stage_body(src_hbm, dst_hbm, f_hbm, s_hbm, d_hbm, lst_hbm, wst_hbm,
                     idx_s, idx_d, rowsL, wbuf, sv, dv, sem):
    _, _, ebase = _wid_base()
    pltpu.sync_copy(s_hbm, sv)
    pltpu.sync_copy(d_hbm, dv)

    def chunk(ci, _):
        eb = ebase + ci * K
        pltpu.sync_copy(src_hbm.at[pl.ds(eb, K)], idx_s)
        pltpu.sync_copy(dst_hbm.at[pl.ds(eb, K)], idx_d)
        cl = pltpu.async_copy(f_hbm.at[idx_s], rowsL, sem)

        def wg(g, _):
            sidx = idx_s[pl.ds(g * 16, 16)]
            didx = idx_d[pl.ds(g * 16, 16)]
            t = plsc.load_gather(sv, [sidx]) + plsc.load_gather(dv, [didx])
            wbuf[pl.ds(g * 16, 16)] = jnp.exp(_leaky(t))
            return 0
        lax.fori_loop(0, K // 16, wg, 0)

        cl.wait()
        pltpu.sync_copy(rowsL, lst_hbm.at[pl.ds(eb, K), :])
        pltpu.sync_copy(wbuf, wst_hbm.at[pl.ds(eb, K)])
        return 0
    lax.fori_loop(0, NCHUNK, chunk, 0)


def _gat1_stage_pass(src, dst, feat, s, d):
    return pl.kernel(
        _gat1_stage_body,
        out_type=(jax.ShapeDtypeStruct((E, H), F32),
                  jax.ShapeDtypeStruct((E,), F32)),
        mesh=_MESH,
        scratch_types=[pltpu.VMEM((K,), I32), pltpu.VMEM((K,), I32),
                       pltpu.VMEM((K, H), F32), pltpu.VMEM((K,), F32),
                       pltpu.VMEM((N,), F32), pltpu.VMEM((N,), F32),
                       pltpu.SemaphoreType.DMA],
        compiler_params=_SC_PARAMS,
    )(src, dst, feat, s, d)


def _zero_accumulators(sid, r0, zbuf, zs, sharedU, sharedS):
    zero16 = jnp.zeros((16,), F32)
    for r in range(ZR):
        for j in range(H // 16):
            zbuf[r, pl.ds(j * 16, 16)] = zero16

    @pl.when(sid < NWRITE)
    def _zero_u():
        def zcp(q, _):
            pltpu.sync_copy(zbuf, sharedU.at[pl.ds(r0 + q * ZR, ZR), :])
            return 0
        lax.fori_loop(0, RW // ZR, zcp, 0)

    if sharedS is not None:
        def zs_(i, _):
            zs[pl.ds(i * 16, 16)] = zero16
            return 0
        lax.fori_loop(0, 125, zs_, 0)

        @pl.when(sid == 0)
        def _zero_s():
            def scp(q, _):
                pltpu.sync_copy(zs, sharedS.at[pl.ds(q * 2000, 2000)])
                return 0
            lax.fori_loop(0, N // 2000, scp, 0)


def _scatter_body(dst_hbm, rows_hbm, w_hbm, outU, outS,
                  idx_d, rowsL, wbuf, zbuf, zs, sharedU, sharedS):
    cid, sid, ebase = _wid_base()
    r0 = pl.multiple_of(sid * RW, 8)
    _zero_accumulators(sid, r0, zbuf, zs, sharedU, sharedS)
    plsc.subcore_barrier()

    def chunk(ci, _):
        eb = ebase + ci * K
        pltpu.sync_copy(dst_hbm.at[pl.ds(eb, K)], idx_d)
        pltpu.sync_copy(rows_hbm.at[pl.ds(eb, K), :], rowsL)
        pltpu.sync_copy(w_hbm.at[pl.ds(eb, K)], wbuf)
        pltpu.sync_copy(rowsL, sharedU.at[idx_d], add=True)
        pltpu.sync_copy(wbuf, sharedS.at[idx_d], add=True)
        return 0
    lax.fori_loop(0, NCHUNK, chunk, 0)

    plsc.subcore_barrier()

    @pl.when(sid < NWRITE)
    def _write_u():
        pltpu.sync_copy(sharedU.at[pl.ds(r0, RW), :],
                        outU.at[cid, pl.ds(r0, RW), :])

    @pl.when(sid == 0)
    def _write_s():
        pltpu.sync_copy(sharedS, outS.at[cid])


def _scatter_pass(dst, rows, w):
    return pl.kernel(
        _scatter_body,
        out_type=(jax.ShapeDtypeStruct((NCORES, N, H), F32),
                  jax.ShapeDtypeStruct((NCORES, N), F32)),
        mesh=_MESH,
        scratch_types=[pltpu.VMEM((K,), I32), pltpu.VMEM((K, H), F32),
                       pltpu.VMEM((K,), F32), pltpu.VMEM((ZR, H), F32),
                       pltpu.VMEM((2000,), F32),
                       pltpu.VMEM_SHARED((N, H), F32),
                       pltpu.VMEM_SHARED((N,), F32)],
        compiler_params=_SC_PARAMS,
    )(dst, rows, w)


def _conv_body(src_hbm, dst_hbm, f_hbm, outU,
               idx_s, idx_d, rowsL, zbuf, sharedU, sem):
    cid, sid, ebase = _wid_base()
    r0 = pl.multiple_of(sid * RW, 8)
    _zero_accumulators(sid, r0, zbuf, None, sharedU, None)
    plsc.subcore_barrier()

    def chunk(ci, _):
        eb = ebase + ci * K
        pltpu.sync_copy(src_hbm.at[pl.ds(eb, K)], idx_s)
        pltpu.sync_copy(dst_hbm.at[pl.ds(eb, K)], idx_d)
        pltpu.async_copy(f_hbm.at[idx_s], rowsL, sem).wait()
        pltpu.sync_copy(rowsL, sharedU.at[idx_d], add=True)
        return 0
    lax.fori_loop(0, NCHUNK, chunk, 0)

    plsc.subcore_barrier()

    @pl.when(sid < NWRITE)
    def _write_u():
        pltpu.sync_copy(sharedU.at[pl.ds(r0, RW), :],
                        outU.at[cid, pl.ds(r0, RW), :])


def _conv_pass(src, dst, feat):
    return pl.kernel(
        _conv_body,
        out_type=jax.ShapeDtypeStruct((NCORES, N, H), F32),
        mesh=_MESH,
        scratch_types=[pltpu.VMEM((K,), I32), pltpu.VMEM((K,), I32),
                       pltpu.VMEM((K, H), F32), pltpu.VMEM((ZR, H), F32),
                       pltpu.VMEM_SHARED((N, H), F32),
                       pltpu.SemaphoreType.DMA],
        compiler_params=_SC_PARAMS,
    )(src, dst, feat)


def kernel(x, cat_features, edge_index, batch, W1, a_src1, a_dst1, b1,
           W_rel, b_rel, W_root, Wl3, bl3, Wr3, br3, att3, b3,
           Wl4, bl4, Wr4, br4, att4, b4, Wfc1, bfc1, Wfc2, bfc2):
    src = edge_index[0]
    dst = edge_index[1]

    sN = jax.ShapeDtypeStruct((N, H), F32)
    s1 = jax.ShapeDtypeStruct((N, 1), F32)

    xl, s, d = _tc(_tc1_body, [sN, s1, s1], x, W1, a_src1, a_dst1)
    L1, w1e = _gat1_stage_pass(src, dst, xl, s.reshape(N), d.reshape(N))
    wL1 = _tc_scale_gat1(L1, w1e.reshape(E, 1))
    U1, S1 = _scatter_pass(dst, wL1, w1e)
    h1 = _tc(_tc2_body, sN, U1, S1.reshape(NCORES, N, 1), xl, s, d, b1)
    A2 = _conv_pass(src, dst, h1)
    h2, zl3, zr3, w3 = _tc(
        _tc3_body, [sN, sN, sN, s1],
        A2, h1, W_rel, b_rel, W_root, Wl3, bl3, Wr3, br3, att3.reshape(1, H))
    L3, R3 = _gather2_pass(src, dst, zl3, zr3)
    wL3, w3e = _tc_scale_gatv2(L3, R3, att3.reshape(1, H))
    U3, S3 = _scatter_pass(dst, wL3, w3e.reshape(E))
    zl4, zr4, w4 = _tc(
        _tc4_body, [sN, sN, s1],
        U3, S3.reshape(NCORES, N, 1), zl3, w3, b3, h2,
        Wl4, bl4, Wr4, br4, att4.reshape(1, H))
    L4, R4 = _gather2_pass(src, dst, zl4, zr4)
    wL4, w4e = _tc_scale_gatv2(L4, R4, att4.reshape(1, H))
    U4, S4 = _scatter_pass(dst, wL4, w4e.reshape(E))
    y = _tc(
        _tc5_body, jax.ShapeDtypeStruct((NG, 1), F32),
        U4, S4.reshape(NCORES, N, 1), zl4, w4, b4, batch.reshape(N, 1),
        cat_features, Wfc1, bfc1, Wfc2, bfc2)
    return y


# speedup vs baseline: 12.3892x; 1.5830x over previous
import jax
import jax.numpy as jnp
from jax import lax
from jax.experimental import pallas as pl
from jax.experimental.pallas import tpu as pltpu
from jax.experimental.pallas import tpu_sc as plsc

N = 10000
E = 320000
H = 128
NCAT = 16
NG = 64
FCH = 600

NCORES = 2
NSUB = 16
NW = NCORES * NSUB
EPT = E // NW
K = 80
NCHUNK = EPT // K
NWRITE = 10
RW = N // NWRITE
ZR = 40
BE = 8000
F32 = jnp.float32
I32 = jnp.int32


def _leaky(t):
    return jnp.where(t > 0, t, 0.2 * t)


def _tc1_body(x_ref, w1_ref, asrc_ref, adst_ref, xl_ref, s_ref, d_ref):
    xl = jnp.dot(x_ref[...], w1_ref[...], preferred_element_type=F32)
    xl_ref[...] = xl
    s_ref[...] = jnp.sum(xl * asrc_ref[...][None, :], axis=1, keepdims=True)
    d_ref[...] = jnp.sum(xl * adst_ref[...][None, :], axis=1, keepdims=True)


def _tc2_body(u_ref, sa_ref, xl_ref, s_ref, d_ref, b1_ref, h_ref):
    w = jnp.exp(_leaky(s_ref[...] + d_ref[...]))
    xl = xl_ref[...]
    U = u_ref[0] + u_ref[1] + w * xl
    S = sa_ref[0] + sa_ref[1] + w
    h_ref[...] = jnp.maximum(U / jnp.maximum(S, 1e-16) + b1_ref[...][None, :], 0.0)


def _tc3_body(agg_ref, h1_ref, wrel_ref, brel_ref, wroot_ref,
              wl_ref, bl_ref, wr_ref, br_ref, att_ref,
              h2_ref, zl_ref, zr_ref, w3_ref):
    agg = agg_ref[0] + agg_ref[1]
    h2 = jnp.maximum(
        jnp.dot(agg, wrel_ref[...], preferred_element_type=F32)
        + brel_ref[...][None, :]
        + jnp.dot(h1_ref[...], wroot_ref[...], preferred_element_type=F32), 0.0)
    zl = jnp.dot(h2, wl_ref[...], preferred_element_type=F32) + bl_ref[...][None, :]
    zr = jnp.dot(h2, wr_ref[...], preferred_element_type=F32) + br_ref[...][None, :]
    h2_ref[...] = h2
    zl_ref[...] = zl
    zr_ref[...] = zr
    w3_ref[...] = jnp.exp(jnp.sum(_leaky(zl + zr) * att_ref[...],
                                  axis=1, keepdims=True))


def _tc4_body(u_ref, sa_ref, zl3_ref, w3_ref, b3_ref, h2_ref,
              wl_ref, bl_ref, wr_ref, br_ref, att_ref,
              zl_ref, zr_ref, w4_ref):
    w3 = w3_ref[...]
    U = u_ref[0] + u_ref[1] + w3 * zl3_ref[...]
    S = sa_ref[0] + sa_ref[1] + w3
    h3 = jnp.maximum(U / jnp.maximum(S, 1e-16) + b3_ref[...][None, :], 0.0)
    h = h3 + h2_ref[...]
    zl = jnp.dot(h, wl_ref[...], preferred_element_type=F32) + bl_ref[...][None, :]
    zr = jnp.dot(h, wr_ref[...], preferred_element_type=F32) + br_ref[...][None, :]
    zl_ref[...] = zl
    zr_ref[...] = zr
    w4_ref[...] = jnp.exp(jnp.sum(_leaky(zl + zr) * att_ref[...],
                                  axis=1, keepdims=True))


def _tc5_body(u_ref, sa_ref, zl4_ref, w4_ref, b4_ref, batch_ref, cat_ref,
              wfc1_ref, bfc1_ref, wfc2_ref, bfc2_ref, y_ref):
    w4 = w4_ref[...]
    U = u_ref[0] + u_ref[1] + w4 * zl4_ref[...]
    S = sa_ref[0] + sa_ref[1] + w4
    h = jnp.maximum(U / jnp.maximum(S, 1e-16) + b4_ref[...][None, :], 0.0)
    onehot = (batch_ref[...] == lax.broadcasted_iota(I32, (N, NG), 1)).astype(F32)
    pooled = lax.dot_general(onehot, h, (((0,), (0,)), ((), ())),
                             preferred_element_type=F32)
    z = jnp.concatenate([pooled, cat_ref[...]], axis=1)
    z = jnp.maximum(jnp.dot(z, wfc1_ref[...], preferred_element_type=F32)
                    + bfc1_ref[...][None, :], 0.0)
    y_ref[...] = jnp.dot(z, wfc2_ref[...], preferred_element_type=F32) \
        + bfc2_ref[...][None, :]


def _tc(body, out_shapes, *args):
    return pl.pallas_call(body, out_shape=out_shapes)(*args)


def _tcs_gatv2_body(l_ref, r_ref, att_ref, wl_ref, w_ref):
    L = l_ref[...]
    w = jnp.exp(jnp.sum(_leaky(L + r_ref[...]) * att_ref[...],
                        axis=1, keepdims=True))
    w_ref[...] = w
    wl_ref[...] = w * L


def _tcs_gat1_body(l_ref, w_ref, wl_ref):
    wl_ref[...] = w_ref[...] * l_ref[...]


def _tc_scale_gatv2(Lst, Rst, att):
    blk = lambda i: (i, 0)
    return pl.pallas_call(
        _tcs_gatv2_body,
        grid=(E // BE,),
        in_specs=[pl.BlockSpec((BE, H), blk), pl.BlockSpec((BE, H), blk),
                  pl.BlockSpec((1, H), lambda i: (0, 0))],
        out_specs=[pl.BlockSpec((BE, H), blk), pl.BlockSpec((BE, 1), blk)],
        out_shape=[jax.ShapeDtypeStruct((E, H), F32),
                   jax.ShapeDtypeStruct((E, 1), F32)],
    )(Lst, Rst, att)


def _tc_scale_gat1(Lst, wst):
    blk = lambda i: (i, 0)
    return pl.pallas_call(
        _tcs_gat1_body,
        grid=(E // BE,),
        in_specs=[pl.BlockSpec((BE, H), blk), pl.BlockSpec((BE, 1), blk)],
        out_specs=pl.BlockSpec((BE, H), blk),
        out_shape=jax.ShapeDtypeStruct((E, H), F32),
    )(Lst, wst)


_MESH = plsc.VectorSubcoreMesh(core_axis_name="c", subcore_axis_name="s")
_SC_PARAMS = pltpu.CompilerParams(needs_layout_passes=False)

NPAIR = (NCHUNK - 1) // 2


def _wid_base():
    cid = lax.axis_index("c")
    sid = lax.axis_index("s")
    return cid, sid, (sid * NCORES + cid) * EPT


def _gather2_body(src_hbm, dst_hbm, fl_hbm, fr_hbm, lst_hbm, rst_hbm,
                  idx_s, idx_d, rowsL, rowsR, semA, semB):
    _, _, ebase = _wid_base()
    sems = (semA, semB)

    def issue(g, b):
        eb = ebase + g * K
        pltpu.sync_copy(src_hbm.at[pl.ds(eb, K)], idx_s.at[b])
        pltpu.sync_copy(dst_hbm.at[pl.ds(eb, K)], idx_d.at[b])
        pltpu.async_copy(fl_hbm.at[idx_s.at[b]], rowsL.at[b], sems[b])
        pltpu.async_copy(fr_hbm.at[idx_d.at[b]], rowsR.at[b], sems[b])

    def drain(b):
        pltpu.make_async_copy(fl_hbm.at[pl.ds(0, K), :], rowsL.at[b], sems[b]).wait()
        pltpu.make_async_copy(fr_hbm.at[pl.ds(0, K), :], rowsR.at[b], sems[b]).wait()

    def writeback(g, b):
        eb = ebase + g * K
        pltpu.sync_copy(rowsL.at[b], lst_hbm.at[pl.ds(eb, K), :])
        pltpu.sync_copy(rowsR.at[b], rst_hbm.at[pl.ds(eb, K), :])

    issue(0, 0)

    def pair(p, _):
        g0 = 2 * p
        issue(g0 + 1, 1)
        drain(0)
        writeback(g0, 0)
        issue(g0 + 2, 0)
        drain(1)
        writeback(g0 + 1, 1)
        return 0
    lax.fori_loop(0, NPAIR, pair, 0)
    drain(0)
    writeback(NCHUNK - 1, 0)


def _gather2_pass(src, dst, featL, featR):
    return pl.kernel(
        _gather2_body,
        out_type=(jax.ShapeDtypeStruct((E, H), F32),
                  jax.ShapeDtypeStruct((E, H), F32)),
        mesh=_MESH,
        scratch_types=[pltpu.VMEM((2, K), I32), pltpu.VMEM((2, K), I32),
                       pltpu.VMEM((2, K, H), F32), pltpu.VMEM((2, K, H), F32),
                       pltpu.SemaphoreType.DMA, pltpu.SemaphoreType.DMA],
        compiler_params=_SC_PARAMS,
    )(src, dst, featL, featR)


def _gat1_stage_body(src_hbm, dst_hbm, f_hbm, s_hbm, d_hbm, lst_hbm, wst_hbm,
                     idx_s, idx_d, rowsL, wbuf, sv, dv, semA, semB):
    _, _, ebase = _wid_base()
    sems = (semA, semB)
    pltpu.sync_copy(s_hbm, sv)
    pltpu.sync_copy(d_hbm, dv)

    def issue(g, b):
        eb = ebase + g * K
        pltpu.sync_copy(src_hbm.at[pl.ds(eb, K)], idx_s.at[b])
        pltpu.sync_copy(dst_hbm.at[pl.ds(eb, K)], idx_d.at[b])
        pltpu.async_copy(f_hbm.at[idx_s.at[b]], rowsL.at[b], sems[b])

    def finish(g, b):
        def wg(q, _):
            sidx = idx_s[b, pl.ds(q * 16, 16)]
            didx = idx_d[b, pl.ds(q * 16, 16)]
            t = plsc.load_gather(sv, [sidx]) + plsc.load_gather(dv, [didx])
            wbuf[pl.ds(q * 16, 16)] = jnp.exp(_leaky(t))
            return 0
        lax.fori_loop(0, K // 16, wg, 0)
        eb = ebase + g * K
        pltpu.sync_copy(wbuf, wst_hbm.at[pl.ds(eb, K)])
        pltpu.make_async_copy(f_hbm.at[pl.ds(0, K), :], rowsL.at[b], sems[b]).wait()
        pltpu.sync_copy(rowsL.at[b], lst_hbm.at[pl.ds(eb, K), :])

    issue(0, 0)

    def pair(p, _):
        g0 = 2 * p
        issue(g0 + 1, 1)
        finish(g0, 0)
        issue(g0 + 2, 0)
        finish(g0 + 1, 1)
        return 0
    lax.fori_loop(0, NPAIR, pair, 0)
    finish(NCHUNK - 1, 0)


def _gat1_stage_pass(src, dst, feat, s, d):
    return pl.kernel(
        _gat1_stage_body,
        out_type=(jax.ShapeDtypeStruct((E, H), F32),
                  jax.ShapeDtypeStruct((E,), F32)),
        mesh=_MESH,
        scratch_types=[pltpu.VMEM((2, K), I32), pltpu.VMEM((2, K), I32),
                       pltpu.VMEM((2, K, H), F32), pltpu.VMEM((K,), F32),
                       pltpu.VMEM((N,), F32), pltpu.VMEM((N,), F32),
                       pltpu.SemaphoreType.DMA, pltpu.SemaphoreType.DMA],
        compiler_params=_SC_PARAMS,
    )(src, dst, feat, s, d)


def _zero_accumulators(sid, r0, zbuf, zs, sharedU, sharedS):
    zero16 = jnp.zeros((16,), F32)
    for r in range(ZR):
        for j in range(H // 16):
            zbuf[r, pl.ds(j * 16, 16)] = zero16

    @pl.when(sid < NWRITE)
    def _zero_u():
        def zcp(q, _):
            pltpu.sync_copy(zbuf, sharedU.at[pl.ds(r0 + q * ZR, ZR), :])
            return 0
        lax.fori_loop(0, RW // ZR, zcp, 0)

    if sharedS is not None:
        def zs_(i, _):
            zs[pl.ds(i * 16, 16)] = zero16
            return 0
        lax.fori_loop(0, 125, zs_, 0)

        @pl.when(sid == 0)
        def _zero_s():
            def scp(q, _):
                pltpu.sync_copy(zs, sharedS.at[pl.ds(q * 2000, 2000)])
                return 0
            lax.fori_loop(0, N // 2000, scp, 0)


def _scatter_body(dst_hbm, rows_hbm, w_hbm, outU, outS,
                  idx_d, rowsL, wbuf, zbuf, zs, sharedU, sharedS, semA, semB):
    cid, sid, ebase = _wid_base()
    sems = (semA, semB)
    r0 = pl.multiple_of(sid * RW, 8)
    _zero_accumulators(sid, r0, zbuf, zs, sharedU, sharedS)
    plsc.subcore_barrier()

    def issue(g, b):
        eb = ebase + g * K
        pltpu.async_copy(dst_hbm.at[pl.ds(eb, K)], idx_d.at[b], sems[b])
        pltpu.async_copy(rows_hbm.at[pl.ds(eb, K), :], rowsL.at[b], sems[b])
        pltpu.async_copy(w_hbm.at[pl.ds(eb, K)], wbuf.at[b], sems[b])

    def finish(b):
        pltpu.make_async_copy(dst_hbm.at[pl.ds(0, K)], idx_d.at[b], sems[b]).wait()
        pltpu.make_async_copy(rows_hbm.at[pl.ds(0, K), :], rowsL.at[b], sems[b]).wait()
        pltpu.make_async_copy(w_hbm.at[pl.ds(0, K)], wbuf.at[b], sems[b]).wait()
        pltpu.sync_copy(rowsL.at[b], sharedU.at[idx_d.at[b]], add=True)
        pltpu.sync_copy(wbuf.at[b], sharedS.at[idx_d.at[b]], add=True)

    issue(0, 0)

    def pair(p, _):
        issue(2 * p + 1, 1)
        finish(0)
        issue(2 * p + 2, 0)
        finish(1)
        return 0
    lax.fori_loop(0, NPAIR, pair, 0)
    finish(0)

    plsc.subcore_barrier()

    @pl.when(sid < NWRITE)
    def _write_u():
        pltpu.sync_copy(sharedU.at[pl.ds(r0, RW), :],
                        outU.at[cid, pl.ds(r0, RW), :])

    @pl.when(sid == 0)
    def _write_s():
        pltpu.sync_copy(sharedS, outS.at[cid])


def _scatter_pass(dst, rows, w):
    return pl.kernel(
        _scatter_body,
        out_type=(jax.ShapeDtypeStruct((NCORES, N, H), F32),
                  jax.ShapeDtypeStruct((NCORES, N), F32)),
        mesh=_MESH,
        scratch_types=[pltpu.VMEM((2, K), I32), pltpu.VMEM((2, K, H), F32),
                       pltpu.VMEM((2, K), F32), pltpu.VMEM((ZR, H), F32),
                       pltpu.VMEM((2000,), F32),
                       pltpu.VMEM_SHARED((N, H), F32),
                       pltpu.VMEM_SHARED((N,), F32),
                       pltpu.SemaphoreType.DMA, pltpu.SemaphoreType.DMA],
        compiler_params=_SC_PARAMS,
    )(dst, rows, w)


def _conv_body(src_hbm, dst_hbm, f_hbm, outU,
               idx_s, idx_d, rowsL, zbuf, sharedU, semA, semB):
    cid, sid, ebase = _wid_base()
    sems = (semA, semB)
    r0 = pl.multiple_of(sid * RW, 8)
    _zero_accumulators(sid, r0, zbuf, None, sharedU, None)
    plsc.subcore_barrier()

    def issue(g, b):
        eb = ebase + g * K
        pltpu.sync_copy(src_hbm.at[pl.ds(eb, K)], idx_s.at[b])
        pltpu.sync_copy(dst_hbm.at[pl.ds(eb, K)], idx_d.at[b])
        pltpu.async_copy(f_hbm.at[idx_s.at[b]], rowsL.at[b], sems[b])

    def finish(b):
        pltpu.make_async_copy(f_hbm.at[pl.ds(0, K), :], rowsL.at[b], sems[b]).wait()
        pltpu.sync_copy(rowsL.at[b], sharedU.at[idx_d.at[b]], add=True)

    issue(0, 0)

    def pair(p, _):
        issue(2 * p + 1, 1)
        finish(0)
        issue(2 * p + 2, 0)
        finish(1)
        return 0
    lax.fori_loop(0, NPAIR, pair, 0)
    finish(0)

    plsc.subcore_barrier()

    @pl.when(sid < NWRITE)
    def _write_u():
        pltpu.sync_copy(sharedU.at[pl.ds(r0, RW), :],
                        outU.at[cid, pl.ds(r0, RW), :])


def _conv_pass(src, dst, feat):
    return pl.kernel(
        _conv_body,
        out_type=jax.ShapeDtypeStruct((NCORES, N, H), F32),
        mesh=_MESH,
        scratch_types=[pltpu.VMEM((2, K), I32), pltpu.VMEM((2, K), I32),
                       pltpu.VMEM((2, K, H), F32), pltpu.VMEM((ZR, H), F32),
                       pltpu.VMEM_SHARED((N, H), F32),
                       pltpu.SemaphoreType.DMA, pltpu.SemaphoreType.DMA],
        compiler_params=_SC_PARAMS,
    )(src, dst, feat)


def kernel(x, cat_features, edge_index, batch, W1, a_src1, a_dst1, b1,
           W_rel, b_rel, W_root, Wl3, bl3, Wr3, br3, att3, b3,
           Wl4, bl4, Wr4, br4, att4, b4, Wfc1, bfc1, Wfc2, bfc2):
    src = edge_index[0]
    dst = edge_index[1]

    sN = jax.ShapeDtypeStruct((N, H), F32)
    s1 = jax.ShapeDtypeStruct((N, 1), F32)

    xl, s, d = _tc(_tc1_body, [sN, s1, s1], x, W1, a_src1, a_dst1)
    L1, w1e = _gat1_stage_pass(src, dst, xl, s.reshape(N), d.reshape(N))
    wL1 = _tc_scale_gat1(L1, w1e.reshape(E, 1))
    U1, S1 = _scatter_pass(dst, wL1, w1e)
    h1 = _tc(_tc2_body, sN, U1, S1.reshape(NCORES, N, 1), xl, s, d, b1)
    A2 = _conv_pass(src, dst, h1)
    h2, zl3, zr3, w3 = _tc(
        _tc3_body, [sN, sN, sN, s1],
        A2, h1, W_rel, b_rel, W_root, Wl3, bl3, Wr3, br3, att3.reshape(1, H))
    L3, R3 = _gather2_pass(src, dst, zl3, zr3)
    wL3, w3e = _tc_scale_gatv2(L3, R3, att3.reshape(1, H))
    U3, S3 = _scatter_pass(dst, wL3, w3e.reshape(E))
    zl4, zr4, w4 = _tc(
        _tc4_body, [sN, sN, s1],
        U3, S3.reshape(NCORES, N, 1), zl3, w3, b3, h2,
        Wl4, bl4, Wr4, br4, att4.reshape(1, H))
    L4, R4 = _gather2_pass(src, dst, zl4, zr4)
    wL4, w4e = _tc_scale_gatv2(L4, R4, att4.reshape(1, H))
    U4, S4 = _scatter_pass(dst, wL4, w4e.reshape(E))
    y = _tc(
        _tc5_body, jax.ShapeDtypeStruct((NG, 1), F32),
        U4, S4.reshape(NCORES, N, 1), zl4, w4, b4, batch.reshape(N, 1),
        cat_features, Wfc1, bfc1, Wfc2, bfc2)
    return y


# speedup vs baseline: 13.5013x; 1.0898x over previous
import jax
import jax.numpy as jnp
from jax import lax
from jax.experimental import pallas as pl
from jax.experimental.pallas import tpu as pltpu
from jax.experimental.pallas import tpu_sc as plsc

N = 10000
E = 320000
H = 128
NCAT = 16
NG = 64
FCH = 600

NCORES = 2
NSUB = 16
NW = NCORES * NSUB
EPT = E // NW
K = 80
NCHUNK = EPT // K
NWRITE = 10
RW = N // NWRITE
ZR = 40
BE = 8000
F32 = jnp.float32
I32 = jnp.int32


def _leaky(t):
    return jnp.where(t > 0, t, 0.2 * t)


def _tc1_body(x_ref, w1_ref, asrc_ref, adst_ref, xl_ref, s_ref, d_ref):
    xl = jnp.dot(x_ref[...], w1_ref[...], preferred_element_type=F32)
    xl_ref[...] = xl
    s_ref[...] = jnp.sum(xl * asrc_ref[...][None, :], axis=1, keepdims=True)
    d_ref[...] = jnp.sum(xl * adst_ref[...][None, :], axis=1, keepdims=True)


def _tc2_body(u_ref, sa_ref, xl_ref, s_ref, d_ref, b1_ref, h_ref):
    w = jnp.exp(_leaky(s_ref[...] + d_ref[...]))
    xl = xl_ref[...]
    U = u_ref[0] + u_ref[1] + w * xl
    S = sa_ref[0] + sa_ref[1] + w
    h_ref[...] = jnp.maximum(U / jnp.maximum(S, 1e-16) + b1_ref[...][None, :], 0.0)


def _tc3_body(agg_ref, h1_ref, wrel_ref, brel_ref, wroot_ref,
              wl_ref, bl_ref, wr_ref, br_ref, att_ref,
              h2_ref, zl_ref, zr_ref, w3_ref):
    agg = agg_ref[0] + agg_ref[1]
    h2 = jnp.maximum(
        jnp.dot(agg, wrel_ref[...], preferred_element_type=F32)
        + brel_ref[...][None, :]
        + jnp.dot(h1_ref[...], wroot_ref[...], preferred_element_type=F32), 0.0)
    zl = jnp.dot(h2, wl_ref[...], preferred_element_type=F32) + bl_ref[...][None, :]
    zr = jnp.dot(h2, wr_ref[...], preferred_element_type=F32) + br_ref[...][None, :]
    h2_ref[...] = h2
    zl_ref[...] = zl
    zr_ref[...] = zr
    w3_ref[...] = jnp.exp(jnp.sum(_leaky(zl + zr) * att_ref[...],
                                  axis=1, keepdims=True))


def _tc4_body(u_ref, sa_ref, zl3_ref, w3_ref, b3_ref, h2_ref,
              wl_ref, bl_ref, wr_ref, br_ref, att_ref,
              zl_ref, zr_ref, w4_ref):
    w3 = w3_ref[...]
    U = u_ref[0] + u_ref[1] + w3 * zl3_ref[...]
    S = sa_ref[0] + sa_ref[1] + w3
    h3 = jnp.maximum(U / jnp.maximum(S, 1e-16) + b3_ref[...][None, :], 0.0)
    h = h3 + h2_ref[...]
    zl = jnp.dot(h, wl_ref[...], preferred_element_type=F32) + bl_ref[...][None, :]
    zr = jnp.dot(h, wr_ref[...], preferred_element_type=F32) + br_ref[...][None, :]
    zl_ref[...] = zl
    zr_ref[...] = zr
    w4_ref[...] = jnp.exp(jnp.sum(_leaky(zl + zr) * att_ref[...],
                                  axis=1, keepdims=True))


def _tc5_body(u_ref, sa_ref, zl4_ref, w4_ref, b4_ref, batch_ref, cat_ref,
              wfc1_ref, bfc1_ref, wfc2_ref, bfc2_ref, y_ref):
    w4 = w4_ref[...]
    U = u_ref[0] + u_ref[1] + w4 * zl4_ref[...]
    S = sa_ref[0] + sa_ref[1] + w4
    h = jnp.maximum(U / jnp.maximum(S, 1e-16) + b4_ref[...][None, :], 0.0)
    onehot = (batch_ref[...] == lax.broadcasted_iota(I32, (N, NG), 1)).astype(F32)
    pooled = lax.dot_general(onehot, h, (((0,), (0,)), ((), ())),
                             preferred_element_type=F32)
    z = jnp.concatenate([pooled, cat_ref[...]], axis=1)
    z = jnp.maximum(jnp.dot(z, wfc1_ref[...], preferred_element_type=F32)
                    + bfc1_ref[...][None, :], 0.0)
    y_ref[...] = jnp.dot(z, wfc2_ref[...], preferred_element_type=F32) \
        + bfc2_ref[...][None, :]


def _tc(body, out_shapes, *args):
    return pl.pallas_call(body, out_shape=out_shapes)(*args)


def _tcs_gatv2_body(l_ref, r_ref, att_ref, wl_ref, w_ref):
    L = l_ref[...]
    w = jnp.exp(jnp.sum(_leaky(L + r_ref[...]) * att_ref[...],
                        axis=1, keepdims=True))
    w_ref[...] = w
    wl_ref[...] = w * L


def _tcs_gat1_body(l_ref, w_ref, wl_ref):
    wl_ref[...] = w_ref[...] * l_ref[...]


def _tc_scale_gatv2(Lst, Rst, att):
    blk = lambda i: (i, 0)
    return pl.pallas_call(
        _tcs_gatv2_body,
        grid=(E // BE,),
        in_specs=[pl.BlockSpec((BE, H), blk), pl.BlockSpec((BE, H), blk),
                  pl.BlockSpec((1, H), lambda i: (0, 0))],
        out_specs=[pl.BlockSpec((BE, H), blk), pl.BlockSpec((BE, 1), blk)],
        out_shape=[jax.ShapeDtypeStruct((E, H), F32),
                   jax.ShapeDtypeStruct((E, 1), F32)],
    )(Lst, Rst, att)


def _tc_scale_gat1(Lst, wst):
    blk = lambda i: (i, 0)
    return pl.pallas_call(
        _tcs_gat1_body,
        grid=(E // BE,),
        in_specs=[pl.BlockSpec((BE, H), blk), pl.BlockSpec((BE, 1), blk)],
        out_specs=pl.BlockSpec((BE, H), blk),
        out_shape=jax.ShapeDtypeStruct((E, H), F32),
    )(Lst, wst)


_MESH = plsc.VectorSubcoreMesh(core_axis_name="c", subcore_axis_name="s")
_SC_PARAMS = pltpu.CompilerParams(needs_layout_passes=False)

NPAIR = (NCHUNK - 1) // 2


def _wid_base():
    cid = lax.axis_index("c")
    sid = lax.axis_index("s")
    return cid, sid, (sid * NCORES + cid) * EPT


def _gather2_body(src_hbm, dst_hbm, fl_hbm, fr_hbm, lst_hbm, rst_hbm,
                  idx_s, idx_d, rowsL, rowsR, semA, semB):
    cid, sid, ebase = _wid_base()
    wid = sid * NCORES + cid
    sems = (semA, semB)
    pltpu.sync_copy(src_hbm.at[wid], idx_s)
    pltpu.sync_copy(dst_hbm.at[wid], idx_d)

    def issue(g, b):
        pltpu.async_copy(fl_hbm.at[idx_s.at[g]], rowsL.at[b], sems[b])
        pltpu.async_copy(fr_hbm.at[idx_d.at[g]], rowsR.at[b], sems[b])

    def drain(b):
        pltpu.make_async_copy(fl_hbm.at[pl.ds(0, K), :], rowsL.at[b], sems[b]).wait()
        pltpu.make_async_copy(fr_hbm.at[pl.ds(0, K), :], rowsR.at[b], sems[b]).wait()

    def writeback(g, b):
        eb = ebase + g * K
        pltpu.sync_copy(rowsL.at[b], lst_hbm.at[pl.ds(eb, K), :])
        pltpu.sync_copy(rowsR.at[b], rst_hbm.at[pl.ds(eb, K), :])

    issue(0, 0)

    def pair(p, _):
        g0 = 2 * p
        issue(g0 + 1, 1)
        drain(0)
        writeback(g0, 0)
        issue(g0 + 2, 0)
        drain(1)
        writeback(g0 + 1, 1)
        return 0
    lax.fori_loop(0, NPAIR, pair, 0)
    drain(0)
    writeback(NCHUNK - 1, 0)


def _gather2_pass(src, dst, featL, featR):
    return pl.kernel(
        _gather2_body,
        out_type=(jax.ShapeDtypeStruct((E, H), F32),
                  jax.ShapeDtypeStruct((E, H), F32)),
        mesh=_MESH,
        scratch_types=[pltpu.VMEM((NCHUNK, K), I32), pltpu.VMEM((NCHUNK, K), I32),
                       pltpu.VMEM((2, K, H), F32), pltpu.VMEM((2, K, H), F32),
                       pltpu.SemaphoreType.DMA, pltpu.SemaphoreType.DMA],
        compiler_params=_SC_PARAMS,
    )(src, dst, featL, featR)


def _gat1_stage_body(src_hbm, dst_hbm, f_hbm, s_hbm, d_hbm, lst_hbm, wst_hbm,
                     idx_s, idx_d, rowsL, wbuf, sv, dv, semA, semB):
    cid, sid, ebase = _wid_base()
    wid = sid * NCORES + cid
    sems = (semA, semB)
    pltpu.sync_copy(s_hbm, sv)
    pltpu.sync_copy(d_hbm, dv)
    pltpu.sync_copy(src_hbm.at[wid], idx_s)
    pltpu.sync_copy(dst_hbm.at[wid], idx_d)

    def issue(g, b):
        pltpu.async_copy(f_hbm.at[idx_s.at[g]], rowsL.at[b], sems[b])

    def finish(g, b):
        def wg(q, _):
            sidx = idx_s[g, pl.ds(q * 16, 16)]
            didx = idx_d[g, pl.ds(q * 16, 16)]
            t = plsc.load_gather(sv, [sidx]) + plsc.load_gather(dv, [didx])
            wbuf[pl.ds(q * 16, 16)] = jnp.exp(_leaky(t))
            return 0
        lax.fori_loop(0, K // 16, wg, 0)
        eb = ebase + g * K
        pltpu.sync_copy(wbuf, wst_hbm.at[pl.ds(eb, K)])
        pltpu.make_async_copy(f_hbm.at[pl.ds(0, K), :], rowsL.at[b], sems[b]).wait()
        pltpu.sync_copy(rowsL.at[b], lst_hbm.at[pl.ds(eb, K), :])

    issue(0, 0)

    def pair(p, _):
        g0 = 2 * p
        issue(g0 + 1, 1)
        finish(g0, 0)
        issue(g0 + 2, 0)
        finish(g0 + 1, 1)
        return 0
    lax.fori_loop(0, NPAIR, pair, 0)
    finish(NCHUNK - 1, 0)


def _gat1_stage_pass(src, dst, feat, s, d):
    return pl.kernel(
        _gat1_stage_body,
        out_type=(jax.ShapeDtypeStruct((E, H), F32),
                  jax.ShapeDtypeStruct((E,), F32)),
        mesh=_MESH,
        scratch_types=[pltpu.VMEM((NCHUNK, K), I32), pltpu.VMEM((NCHUNK, K), I32),
                       pltpu.VMEM((2, K, H), F32), pltpu.VMEM((K,), F32),
                       pltpu.VMEM((N,), F32), pltpu.VMEM((N,), F32),
                       pltpu.SemaphoreType.DMA, pltpu.SemaphoreType.DMA],
        compiler_params=_SC_PARAMS,
    )(src, dst, feat, s, d)


def _zero_accumulators(sid, r0, zbuf, zs, sharedU, sharedS):
    zero16 = jnp.zeros((16,), F32)
    for r in range(ZR):
        for j in range(H // 16):
            zbuf[r, pl.ds(j * 16, 16)] = zero16

    @pl.when(sid < NWRITE)
    def _zero_u():
        def zcp(q, _):
            pltpu.sync_copy(zbuf, sharedU.at[pl.ds(r0 + q * ZR, ZR), :])
            return 0
        lax.fori_loop(0, RW // ZR, zcp, 0)

    if sharedS is not None:
        def zs_(i, _):
            zs[pl.ds(i * 16, 16)] = zero16
            return 0
        lax.fori_loop(0, 125, zs_, 0)

        @pl.when(sid == 0)
        def _zero_s():
            def scp(q, _):
                pltpu.sync_copy(zs, sharedS.at[pl.ds(q * 2000, 2000)])
                return 0
            lax.fori_loop(0, N // 2000, scp, 0)


def _scatter_body(dst_hbm, rows_hbm, w_hbm, outU, outS,
                  idx_d, rowsL, wbuf, zbuf, zs, sharedU, sharedS, semA, semB):
    cid, sid, ebase = _wid_base()
    wid = sid * NCORES + cid
    sems = (semA, semB)
    r0 = pl.multiple_of(sid * RW, 8)
    _zero_accumulators(sid, r0, zbuf, zs, sharedU, sharedS)
    plsc.subcore_barrier()

    def issue(g, b):
        eb = ebase + g * K
        pltpu.async_copy(dst_hbm.at[wid, g], idx_d.at[b], sems[b])
        pltpu.async_copy(rows_hbm.at[pl.ds(eb, K), :], rowsL.at[b], sems[b])
        pltpu.async_copy(w_hbm.at[pl.ds(eb, K)], wbuf.at[b], sems[b])

    def finish(b):
        pltpu.make_async_copy(dst_hbm.at[0, 0], idx_d.at[b], sems[b]).wait()
        pltpu.make_async_copy(rows_hbm.at[pl.ds(0, K), :], rowsL.at[b], sems[b]).wait()
        pltpu.make_async_copy(w_hbm.at[pl.ds(0, K)], wbuf.at[b], sems[b]).wait()
        pltpu.sync_copy(rowsL.at[b], sharedU.at[idx_d.at[b]], add=True)
        pltpu.sync_copy(wbuf.at[b], sharedS.at[idx_d.at[b]], add=True)

    issue(0, 0)

    def pair(p, _):
        issue(2 * p + 1, 1)
        finish(0)
        issue(2 * p + 2, 0)
        finish(1)
        return 0
    lax.fori_loop(0, NPAIR, pair, 0)
    finish(0)

    plsc.subcore_barrier()

    @pl.when(sid < NWRITE)
    def _write_u():
        pltpu.sync_copy(sharedU.at[pl.ds(r0, RW), :],
                        outU.at[cid, pl.ds(r0, RW), :])

    @pl.when(sid == 0)
    def _write_s():
        pltpu.sync_copy(sharedS, outS.at[cid])


def _scatter_pass(dst, rows, w):
    return pl.kernel(
        _scatter_body,
        out_type=(jax.ShapeDtypeStruct((NCORES, N, H), F32),
                  jax.ShapeDtypeStruct((NCORES, N), F32)),
        mesh=_MESH,
        scratch_types=[pltpu.VMEM((2, K), I32), pltpu.VMEM((2, K, H), F32),
                       pltpu.VMEM((2, K), F32), pltpu.VMEM((ZR, H), F32),
                       pltpu.VMEM((2000,), F32),
                       pltpu.VMEM_SHARED((N, H), F32),
                       pltpu.VMEM_SHARED((N,), F32),
                       pltpu.SemaphoreType.DMA, pltpu.SemaphoreType.DMA],
        compiler_params=_SC_PARAMS,
    )(dst, rows, w)


def _conv_body(src_hbm, dst_hbm, f_hbm, outU,
               idx_s, idx_d, rowsL, zbuf, sharedU, semA, semB):
    cid, sid, ebase = _wid_base()
    wid = sid * NCORES + cid
    sems = (semA, semB)
    r0 = pl.multiple_of(sid * RW, 8)
    _zero_accumulators(sid, r0, zbuf, None, sharedU, None)
    plsc.subcore_barrier()

    def issue(g, b):
        pltpu.sync_copy(src_hbm.at[wid, g], idx_s.at[b])
        pltpu.sync_copy(dst_hbm.at[wid, g], idx_d.at[b])
        pltpu.async_copy(f_hbm.at[idx_s.at[b]], rowsL.at[b], sems[b])

    def finish(b):
        pltpu.make_async_copy(f_hbm.at[pl.ds(0, K), :], rowsL.at[b], sems[b]).wait()
        pltpu.sync_copy(rowsL.at[b], sharedU.at[idx_d.at[b]], add=True)

    issue(0, 0)

    def pair(p, _):
        issue(2 * p + 1, 1)
        finish(0)
        issue(2 * p + 2, 0)
        finish(1)
        return 0
    lax.fori_loop(0, NPAIR, pair, 0)
    finish(0)

    plsc.subcore_barrier()

    @pl.when(sid < NWRITE)
    def _write_u():
        pltpu.sync_copy(sharedU.at[pl.ds(r0, RW), :],
                        outU.at[cid, pl.ds(r0, RW), :])


def _conv_pass(src, dst, feat):
    return pl.kernel(
        _conv_body,
        out_type=jax.ShapeDtypeStruct((NCORES, N, H), F32),
        mesh=_MESH,
        scratch_types=[pltpu.VMEM((2, K), I32), pltpu.VMEM((2, K), I32),
                       pltpu.VMEM((2, K, H), F32), pltpu.VMEM((ZR, H), F32),
                       pltpu.VMEM_SHARED((N, H), F32),
                       pltpu.SemaphoreType.DMA, pltpu.SemaphoreType.DMA],
        compiler_params=_SC_PARAMS,
    )(src, dst, feat)


def kernel(x, cat_features, edge_index, batch, W1, a_src1, a_dst1, b1,
           W_rel, b_rel, W_root, Wl3, bl3, Wr3, br3, att3, b3,
           Wl4, bl4, Wr4, br4, att4, b4, Wfc1, bfc1, Wfc2, bfc2):
    src = edge_index[0].reshape(NW, NCHUNK, K)
    dst = edge_index[1].reshape(NW, NCHUNK, K)

    sN = jax.ShapeDtypeStruct((N, H), F32)
    s1 = jax.ShapeDtypeStruct((N, 1), F32)

    xl, s, d = _tc(_tc1_body, [sN, s1, s1], x, W1, a_src1, a_dst1)
    L1, w1e = _gat1_stage_pass(src, dst, xl, s.reshape(N), d.reshape(N))
    wL1 = _tc_scale_gat1(L1, w1e.reshape(E, 1))
    U1, S1 = _scatter_pass(dst, wL1, w1e)
    h1 = _tc(_tc2_body, sN, U1, S1.reshape(NCORES, N, 1), xl, s, d, b1)
    A2 = _conv_pass(src, dst, h1)
    h2, zl3, zr3, w3 = _tc(
        _tc3_body, [sN, sN, sN, s1],
        A2, h1, W_rel, b_rel, W_root, Wl3, bl3, Wr3, br3, att3.reshape(1, H))
    L3, R3 = _gather2_pass(src, dst, zl3, zr3)
    wL3, w3e = _tc_scale_gatv2(L3, R3, att3.reshape(1, H))
    U3, S3 = _scatter_pass(dst, wL3, w3e.reshape(E))
    zl4, zr4, w4 = _tc(
        _tc4_body, [sN, sN, s1],
        U3, S3.reshape(NCORES, N, 1), zl3, w3, b3, h2,
        Wl4, bl4, Wr4, br4, att4.reshape(1, H))
    L4, R4 = _gather2_pass(src, dst, zl4, zr4)
    wL4, w4e = _tc_scale_gatv2(L4, R4, att4.reshape(1, H))
    U4, S4 = _scatter_pass(dst, wL4, w4e.reshape(E))
    y = _tc(
        _tc5_body, jax.ShapeDtypeStruct((NG, 1), F32),
        U4, S4.reshape(NCORES, N, 1), zl4, w4, b4, batch.reshape(N, 1),
        cat_features, Wfc1, bfc1, Wfc2, bfc2)
    return y


# speedup vs baseline: 15.3654x; 1.1381x over previous
import jax
import jax.numpy as jnp
from jax import lax
from jax.experimental import pallas as pl
from jax.experimental.pallas import tpu as pltpu
from jax.experimental.pallas import tpu_sc as plsc

N = 10000
E = 320000
H = 128
NCAT = 16
NG = 64
FCH = 600

NCORES = 2
NSUB = 16
NW = NCORES * NSUB
EPT = E // NW
K = 80
NCHUNK = EPT // K
NWRITE = 10
RW = N // NWRITE
ZR = 40
BE = 8000
F32 = jnp.float32
I32 = jnp.int32


def _leaky(t):
    return jnp.where(t > 0, t, 0.2 * t)


def _tc1_body(x_ref, w1_ref, asrc_ref, adst_ref, xl_ref, s_ref, d_ref):
    xl = jnp.dot(x_ref[...], w1_ref[...], preferred_element_type=F32)
    xl_ref[...] = xl
    s_ref[...] = jnp.sum(xl * asrc_ref[...][None, :], axis=1, keepdims=True)
    d_ref[...] = jnp.sum(xl * adst_ref[...][None, :], axis=1, keepdims=True)


def _tc2_body(u_ref, sa_ref, xl_ref, s_ref, d_ref, b1_ref, h_ref):
    w = jnp.exp(_leaky(s_ref[...] + d_ref[...]))
    xl = xl_ref[...]
    U = u_ref[0] + u_ref[1] + w * xl
    S = sa_ref[0] + sa_ref[1] + w
    h_ref[...] = jnp.maximum(U / jnp.maximum(S, 1e-16) + b1_ref[...][None, :], 0.0)


def _tc3_body(agg_ref, h1_ref, wrel_ref, brel_ref, wroot_ref,
              wl_ref, bl_ref, wr_ref, br_ref, att_ref,
              h2_ref, zl_ref, zr_ref, w3_ref):
    agg = agg_ref[0] + agg_ref[1]
    h2 = jnp.maximum(
        jnp.dot(agg, wrel_ref[...], preferred_element_type=F32)
        + brel_ref[...][None, :]
        + jnp.dot(h1_ref[...], wroot_ref[...], preferred_element_type=F32), 0.0)
    zl = jnp.dot(h2, wl_ref[...], preferred_element_type=F32) + bl_ref[...][None, :]
    zr = jnp.dot(h2, wr_ref[...], preferred_element_type=F32) + br_ref[...][None, :]
    h2_ref[...] = h2
    zl_ref[...] = zl
    zr_ref[...] = zr
    w3_ref[...] = jnp.exp(jnp.sum(_leaky(zl + zr) * att_ref[...],
                                  axis=1, keepdims=True))


def _tc4_body(u_ref, sa_ref, zl3_ref, w3_ref, b3_ref, h2_ref,
              wl_ref, bl_ref, wr_ref, br_ref, att_ref,
              zl_ref, zr_ref, w4_ref):
    w3 = w3_ref[...]
    U = u_ref[0] + u_ref[1] + w3 * zl3_ref[...]
    S = sa_ref[0] + sa_ref[1] + w3
    h3 = jnp.maximum(U / jnp.maximum(S, 1e-16) + b3_ref[...][None, :], 0.0)
    h = h3 + h2_ref[...]
    zl = jnp.dot(h, wl_ref[...], preferred_element_type=F32) + bl_ref[...][None, :]
    zr = jnp.dot(h, wr_ref[...], preferred_element_type=F32) + br_ref[...][None, :]
    zl_ref[...] = zl
    zr_ref[...] = zr
    w4_ref[...] = jnp.exp(jnp.sum(_leaky(zl + zr) * att_ref[...],
                                  axis=1, keepdims=True))


def _tc5_body(u_ref, sa_ref, zl4_ref, w4_ref, b4_ref, batch_ref, cat_ref,
              wfc1_ref, bfc1_ref, wfc2_ref, bfc2_ref, y_ref):
    w4 = w4_ref[...]
    U = u_ref[0] + u_ref[1] + w4 * zl4_ref[...]
    S = sa_ref[0] + sa_ref[1] + w4
    h = jnp.maximum(U / jnp.maximum(S, 1e-16) + b4_ref[...][None, :], 0.0)
    onehot = (batch_ref[...] == lax.broadcasted_iota(I32, (N, NG), 1)).astype(F32)
    pooled = lax.dot_general(onehot, h, (((0,), (0,)), ((), ())),
                             preferred_element_type=F32)
    z = jnp.concatenate([pooled, cat_ref[...]], axis=1)
    z = jnp.maximum(jnp.dot(z, wfc1_ref[...], preferred_element_type=F32)
                    + bfc1_ref[...][None, :], 0.0)
    y_ref[...] = jnp.dot(z, wfc2_ref[...], preferred_element_type=F32) \
        + bfc2_ref[...][None, :]


def _tc(body, out_shapes, *args):
    return pl.pallas_call(body, out_shape=out_shapes)(*args)


def _tcs_gatv2_body(l_ref, r_ref, att_ref, wl_ref, w_ref):
    L = l_ref[...]
    w = jnp.exp(jnp.sum(_leaky(L + r_ref[...]) * att_ref[...],
                        axis=1, keepdims=True))
    w_ref[...] = w
    wl_ref[...] = w * L


def _tcs_gat1_body(l_ref, w_ref, wl_ref):
    wl_ref[...] = w_ref[...] * l_ref[...]


def _tc_scale_gatv2(Lst, Rst, att):
    blk = lambda i: (i, 0)
    return pl.pallas_call(
        _tcs_gatv2_body,
        grid=(E // BE,),
        in_specs=[pl.BlockSpec((BE, H), blk), pl.BlockSpec((BE, H), blk),
                  pl.BlockSpec((1, H), lambda i: (0, 0))],
        out_specs=[pl.BlockSpec((BE, H), blk), pl.BlockSpec((BE, 1), blk)],
        out_shape=[jax.ShapeDtypeStruct((E, H), F32),
                   jax.ShapeDtypeStruct((E, 1), F32)],
    )(Lst, Rst, att)


def _tc_scale_gat1(Lst, wst):
    blk = lambda i: (i, 0)
    return pl.pallas_call(
        _tcs_gat1_body,
        grid=(E // BE,),
        in_specs=[pl.BlockSpec((BE, H), blk), pl.BlockSpec((BE, 1), blk)],
        out_specs=pl.BlockSpec((BE, H), blk),
        out_shape=jax.ShapeDtypeStruct((E, H), F32),
    )(Lst, wst)


_MESH = plsc.VectorSubcoreMesh(core_axis_name="c", subcore_axis_name="s")
_SC_PARAMS = pltpu.CompilerParams(needs_layout_passes=False)

NPAIR = (NCHUNK - 1) // 2


def _wid_base():
    cid = lax.axis_index("c")
    sid = lax.axis_index("s")
    return cid, sid, (sid * NCORES + cid) * EPT


def _gather2_body(src_hbm, dst_hbm, fl_hbm, fr_hbm, lst_hbm, rst_hbm,
                  idx_s, idx_d, rowsL, rowsR, semA, semB):
    cid, sid, ebase = _wid_base()
    wid = sid * NCORES + cid
    sems = (semA, semB)
    pltpu.sync_copy(src_hbm.at[wid], idx_s)
    pltpu.sync_copy(dst_hbm.at[wid], idx_d)

    def issue(g, b):
        pltpu.async_copy(fl_hbm.at[idx_s.at[g]], rowsL.at[b], sems[b])
        pltpu.async_copy(fr_hbm.at[idx_d.at[g]], rowsR.at[b], sems[b])

    def drain(b):
        pltpu.make_async_copy(fl_hbm.at[pl.ds(0, K), :], rowsL.at[b], sems[b]).wait()
        pltpu.make_async_copy(fr_hbm.at[pl.ds(0, K), :], rowsR.at[b], sems[b]).wait()

    def writeback(g, b):
        eb = ebase + g * K
        pltpu.sync_copy(rowsL.at[b], lst_hbm.at[pl.ds(eb, K), :])
        pltpu.sync_copy(rowsR.at[b], rst_hbm.at[pl.ds(eb, K), :])

    issue(0, 0)

    def pair(p, _):
        g0 = 2 * p
        issue(g0 + 1, 1)
        drain(0)
        writeback(g0, 0)
        issue(g0 + 2, 0)
        drain(1)
        writeback(g0 + 1, 1)
        return 0
    lax.fori_loop(0, NPAIR, pair, 0)
    drain(0)
    writeback(NCHUNK - 1, 0)


def _gather2_pass(src, dst, featL, featR):
    return pl.kernel(
        _gather2_body,
        out_type=(jax.ShapeDtypeStruct((E, H), F32),
                  jax.ShapeDtypeStruct((E, H), F32)),
        mesh=_MESH,
        scratch_types=[pltpu.VMEM((NCHUNK, K), I32), pltpu.VMEM((NCHUNK, K), I32),
                       pltpu.VMEM((2, K, H), F32), pltpu.VMEM((2, K, H), F32),
                       pltpu.SemaphoreType.DMA, pltpu.SemaphoreType.DMA],
        compiler_params=_SC_PARAMS,
    )(src, dst, featL, featR)


def _gat1_w_body(src_hbm, dst_hbm, s_hbm, d_hbm, wst_hbm,
                 idx_s, idx_d, sv, dv, wtab):
    cid, sid, ebase = _wid_base()
    wid = sid * NCORES + cid
    pltpu.sync_copy(s_hbm, sv)
    pltpu.sync_copy(d_hbm, dv)
    pltpu.sync_copy(src_hbm.at[wid], idx_s)
    pltpu.sync_copy(dst_hbm.at[wid], idx_d)

    def chunk(ci, _):
        def wg(q, _):
            sidx = idx_s[ci, pl.ds(q * 16, 16)]
            didx = idx_d[ci, pl.ds(q * 16, 16)]
            t = plsc.load_gather(sv, [sidx]) + plsc.load_gather(dv, [didx])
            wtab[pl.ds(ci * K + q * 16, 16)] = jnp.exp(_leaky(t))
            return 0
        lax.fori_loop(0, K // 16, wg, 0)
        return 0
    lax.fori_loop(0, NCHUNK, chunk, 0)
    pltpu.sync_copy(wtab, wst_hbm.at[pl.ds(ebase, EPT)])


def _gat1_w_pass(src, dst, s, d):
    return pl.kernel(
        _gat1_w_body,
        out_type=jax.ShapeDtypeStruct((E,), F32),
        mesh=_MESH,
        scratch_types=[pltpu.VMEM((NCHUNK, K), I32), pltpu.VMEM((NCHUNK, K), I32),
                       pltpu.VMEM((N,), F32), pltpu.VMEM((N,), F32),
                       pltpu.VMEM((EPT,), F32)],
        compiler_params=_SC_PARAMS,
    )(src, dst, s, d)


def _zero_accumulators(sid, r0, zbuf, zs, sharedU, sharedS):
    zero16 = jnp.zeros((16,), F32)
    for r in range(ZR):
        for j in range(H // 16):
            zbuf[r, pl.ds(j * 16, 16)] = zero16

    @pl.when(sid < NWRITE)
    def _zero_u():
        def zcp(q, _):
            pltpu.sync_copy(zbuf, sharedU.at[pl.ds(r0 + q * ZR, ZR), :])
            return 0
        lax.fori_loop(0, RW // ZR, zcp, 0)

    if sharedS is not None:
        def zs_(i, _):
            zs[pl.ds(i * 16, 16)] = zero16
            return 0
        lax.fori_loop(0, 125, zs_, 0)

        @pl.when(sid == 0)
        def _zero_s():
            def scp(q, _):
                pltpu.sync_copy(zs, sharedS.at[pl.ds(q * 2000, 2000)])
                return 0
            lax.fori_loop(0, N // 2000, scp, 0)


def _scatter_body(dst_hbm, rows_hbm, w_hbm, outU, outS,
                  idx_d, rowsL, wbuf, zbuf, zs, sharedU, sharedS, semA, semB):
    cid, sid, ebase = _wid_base()
    wid = sid * NCORES + cid
    sems = (semA, semB)
    r0 = pl.multiple_of(sid * RW, 8)
    _zero_accumulators(sid, r0, zbuf, zs, sharedU, sharedS)
    plsc.subcore_barrier()

    def issue(g, b):
        eb = ebase + g * K
        pltpu.async_copy(dst_hbm.at[wid, g], idx_d.at[b], sems[b])
        pltpu.async_copy(rows_hbm.at[pl.ds(eb, K), :], rowsL.at[b], sems[b])
        pltpu.async_copy(w_hbm.at[pl.ds(eb, K)], wbuf.at[b], sems[b])

    def finish(b):
        pltpu.make_async_copy(dst_hbm.at[0, 0], idx_d.at[b], sems[b]).wait()
        pltpu.make_async_copy(rows_hbm.at[pl.ds(0, K), :], rowsL.at[b], sems[b]).wait()
        pltpu.make_async_copy(w_hbm.at[pl.ds(0, K)], wbuf.at[b], sems[b]).wait()
        pltpu.sync_copy(rowsL.at[b], sharedU.at[idx_d.at[b]], add=True)
        pltpu.sync_copy(wbuf.at[b], sharedS.at[idx_d.at[b]], add=True)

    issue(0, 0)

    def pair(p, _):
        issue(2 * p + 1, 1)
        finish(0)
        issue(2 * p + 2, 0)
        finish(1)
        return 0
    lax.fori_loop(0, NPAIR, pair, 0)
    finish(0)

    plsc.subcore_barrier()

    @pl.when(sid < NWRITE)
    def _write_u():
        pltpu.sync_copy(sharedU.at[pl.ds(r0, RW), :],
                        outU.at[cid, pl.ds(r0, RW), :])

    @pl.when(sid == 0)
    def _write_s():
        pltpu.sync_copy(sharedS, outS.at[cid])


def _scatter_pass(dst, rows, w):
    return pl.kernel(
        _scatter_body,
        out_type=(jax.ShapeDtypeStruct((NCORES, N, H), F32),
                  jax.ShapeDtypeStruct((NCORES, N), F32)),
        mesh=_MESH,
        scratch_types=[pltpu.VMEM((2, K), I32), pltpu.VMEM((2, K, H), F32),
                       pltpu.VMEM((2, K), F32), pltpu.VMEM((ZR, H), F32),
                       pltpu.VMEM((2000,), F32),
                       pltpu.VMEM_SHARED((N, H), F32),
                       pltpu.VMEM_SHARED((N,), F32),
                       pltpu.SemaphoreType.DMA, pltpu.SemaphoreType.DMA],
        compiler_params=_SC_PARAMS,
    )(dst, rows, w)


def _conv_body(src_hbm, dst_hbm, f_hbm, w_hbm, outU, outS,
               idx_s, idx_d, rowsL, wbuf, zbuf, zs, sharedU, sharedS,
               semA, semB):
    weighted = w_hbm is not None
    cid, sid, ebase = _wid_base()
    wid = sid * NCORES + cid
    sems = (semA, semB)
    r0 = pl.multiple_of(sid * RW, 8)
    _zero_accumulators(sid, r0, zbuf, zs, sharedU, sharedS)
    plsc.subcore_barrier()

    def issue(g, b):
        pltpu.sync_copy(src_hbm.at[wid, g], idx_s.at[b])
        pltpu.sync_copy(dst_hbm.at[wid, g], idx_d.at[b])
        pltpu.async_copy(f_hbm.at[idx_s.at[b]], rowsL.at[b], sems[b])
        if weighted:
            eb = ebase + g * K
            pltpu.async_copy(w_hbm.at[pl.ds(eb, K)], wbuf.at[b], sems[b])

    def finish(b):
        pltpu.make_async_copy(f_hbm.at[pl.ds(0, K), :], rowsL.at[b], sems[b]).wait()
        if weighted:
            pltpu.make_async_copy(w_hbm.at[pl.ds(0, K)], wbuf.at[b], sems[b]).wait()
            def grp(q, _):
                w16 = wbuf[b, pl.ds(q * 16, 16)]
                for j in range(16):
                    wv = jnp.full((16,), w16[j], F32)
                    r = q * 16 + j
                    for jj in range(H // 16):
                        rowsL[b, r, pl.ds(jj * 16, 16)] = \
                            rowsL[b, r, pl.ds(jj * 16, 16)] * wv
                return 0
            lax.fori_loop(0, K // 16, grp, 0)
        pltpu.sync_copy(rowsL.at[b], sharedU.at[idx_d.at[b]], add=True)
        if weighted:
            pltpu.sync_copy(wbuf.at[b], sharedS.at[idx_d.at[b]], add=True)

    issue(0, 0)

    def pair(p, _):
        issue(2 * p + 1, 1)
        finish(0)
        issue(2 * p + 2, 0)
        finish(1)
        return 0
    lax.fori_loop(0, NPAIR, pair, 0)
    finish(0)

    plsc.subcore_barrier()

    @pl.when(sid < NWRITE)
    def _write_u():
        pltpu.sync_copy(sharedU.at[pl.ds(r0, RW), :],
                        outU.at[cid, pl.ds(r0, RW), :])

    if weighted:
        @pl.when(sid == 0)
        def _write_s():
            pltpu.sync_copy(sharedS, outS.at[cid])


def _conv_pass(src, dst, feat):
    def body(src_hbm, dst_hbm, f_hbm, outU,
             idx_s, idx_d, rowsL, zbuf, sharedU, semA, semB):
        _conv_body(src_hbm, dst_hbm, f_hbm, None, outU, None,
                   idx_s, idx_d, rowsL, None, zbuf, None, sharedU, None,
                   semA, semB)
    return pl.kernel(
        body,
        out_type=jax.ShapeDtypeStruct((NCORES, N, H), F32),
        mesh=_MESH,
        scratch_types=[pltpu.VMEM((2, K), I32), pltpu.VMEM((2, K), I32),
                       pltpu.VMEM((2, K, H), F32), pltpu.VMEM((ZR, H), F32),
                       pltpu.VMEM_SHARED((N, H), F32),
                       pltpu.SemaphoreType.DMA, pltpu.SemaphoreType.DMA],
        compiler_params=_SC_PARAMS,
    )(src, dst, feat)


def _gat1_scatter_pass(src, dst, feat, w):
    return pl.kernel(
        _conv_body,
        out_type=(jax.ShapeDtypeStruct((NCORES, N, H), F32),
                  jax.ShapeDtypeStruct((NCORES, N), F32)),
        mesh=_MESH,
        scratch_types=[pltpu.VMEM((2, K), I32), pltpu.VMEM((2, K), I32),
                       pltpu.VMEM((2, K, H), F32), pltpu.VMEM((2, K), F32),
                       pltpu.VMEM((ZR, H), F32), pltpu.VMEM((2000,), F32),
                       pltpu.VMEM_SHARED((N, H), F32),
                       pltpu.VMEM_SHARED((N,), F32),
                       pltpu.SemaphoreType.DMA, pltpu.SemaphoreType.DMA],
        compiler_params=_SC_PARAMS,
    )(src, dst, feat, w)


def kernel(x, cat_features, edge_index, batch, W1, a_src1, a_dst1, b1,
           W_rel, b_rel, W_root, Wl3, bl3, Wr3, br3, att3, b3,
           Wl4, bl4, Wr4, br4, att4, b4, Wfc1, bfc1, Wfc2, bfc2):
    src = edge_index[0].reshape(NW, NCHUNK, K)
    dst = edge_index[1].reshape(NW, NCHUNK, K)

    sN = jax.ShapeDtypeStruct((N, H), F32)
    sNb = jax.ShapeDtypeStruct((N, H), jnp.bfloat16)
    s1 = jax.ShapeDtypeStruct((N, 1), F32)

    xl, s, d = _tc(_tc1_body, [sN, s1, s1], x, W1, a_src1, a_dst1)
    w1e = _gat1_w_pass(src, dst, s.reshape(N), d.reshape(N))
    U1, S1 = _gat1_scatter_pass(src, dst, xl, w1e)
    h1 = _tc(_tc2_body, sN, U1, S1.reshape(NCORES, N, 1), xl, s, d, b1)
    A2 = _conv_pass(src, dst, h1)
    h2, zl3, zr3, w3 = _tc(
        _tc3_body, [sN, sN, sN, s1],
        A2, h1, W_rel, b_rel, W_root, Wl3, bl3, Wr3, br3, att3.reshape(1, H))
    L3, R3 = _gather2_pass(src, dst, zl3, zr3)
    wL3, w3e = _tc_scale_gatv2(L3, R3, att3.reshape(1, H))
    U3, S3 = _scatter_pass(dst, wL3, w3e.reshape(E))
    zl4, zr4, w4 = _tc(
        _tc4_body, [sN, sN, s1],
        U3, S3.reshape(NCORES, N, 1), zl3, w3, b3, h2,
        Wl4, bl4, Wr4, br4, att4.reshape(1, H))
    L4, R4 = _gather2_pass(src, dst, zl4, zr4)
    wL4, w4e = _tc_scale_gatv2(L4, R4, att4.reshape(1, H))
    U4, S4 = _scatter_pass(dst, wL4, w4e.reshape(E))
    y = _tc(
        _tc5_body, jax.ShapeDtypeStruct((NG, 1), F32),
        U4, S4.reshape(NCORES, N, 1), zl4, w4, b4, batch.reshape(N, 1),
        cat_features, Wfc1, bfc1, Wfc2, bfc2)
    return y
